# Initial kernel scaffold; baseline (speedup 1.0000x reference)
import jax
import jax.numpy as jnp
from jax.experimental import pallas as pl


def kernel(x, q_w, k_w, v_w, out_w, out_b, g_w1, g_b1, g_w2, g_b2, g_w3, g_b3, c_w1, c_b1, c_w2, c_b2, alpha_q, beta_q, alpha_k, beta_k, alpha_v, beta_v):
    raise NotImplementedError("write your pallas kernel here")


# speedup vs baseline: 5.0902x; 5.0902x over previous
import jax
import jax.numpy as jnp
from jax.experimental import pallas as pl
from jax.experimental.pallas import tpu as pltpu

B, S, D = 4, 512, 256
H, T_MAX = 8, 20
HEAD_DIM = D // H
SCALE = HEAD_DIM ** -0.5


def _nt(a, b):
    return jax.lax.dot_general(a, b, (((1,), (1,)), ((), ())),
                               preferred_element_type=jnp.float32)


def _fused_kernel(x_ref, qw_ref, kw_ref, vw_ref, ow_ref, ob_ref,
                  gw1_ref, gb1_ref, gw2_ref, gb2_ref, gw3_ref, gb3_ref,
                  cw1_ref, cb1_ref, cw2_ref, cb2_ref,
                  ab_ref,
                  out_ref,
                  scores_ref, vsum_ref, ctx_ref):
    f32 = jnp.float32
    xb = x_ref[0]

    q = _nt(xb, qw_ref[...])
    k = _nt(xb, kw_ref[...])
    v = _nt(xb, vw_ref[...])

    h1 = jnp.maximum(_nt(xb, gw1_ref[...]) + gb1_ref[...], 0.0)
    h2 = jnp.maximum(_nt(h1, gw2_ref[...]) + gb2_ref[...], 0.0)
    g3 = _nt(h2, gw3_ref[...])[:, 0:1] + gb3_ref[0, 0]
    gate = jax.nn.sigmoid(g3)
    hc = jnp.maximum(_nt(xb, cw1_ref[...]) + cb1_ref[...], 0.0)
    c2 = _nt(hc, cw2_ref[...])[:, 0:1] + cb2_ref[0, 0]
    comp = jax.nn.sigmoid(c2)
    combined = 0.7 * gate + 0.3 * comp
    t_lim = jnp.clip(jnp.ceil(combined * T_MAX), 1.0, float(T_MAX))

    alpha_q = ab_ref[0, 0]
    beta_q = ab_ref[0, 1]
    alpha_k = ab_ref[0, 2]
    beta_k = ab_ref[0, 3]
    alpha_v = ab_ref[0, 4]
    beta_v = ab_ref[0, 5]

    scores_ref[...] = jnp.zeros_like(scores_ref)
    vsum_ref[...] = jnp.zeros_like(vsum_ref)

    def step(t, carry):
        vq, iq, vk, ik, vv, iv = carry
        mask = (t.astype(f32) < t_lim).astype(f32)

        iq = alpha_q * iq + q
        vq = beta_q * vq + iq
        sq = (vq >= 1.0).astype(f32)
        vq = vq * (1.0 - sq)

        ik = alpha_k * ik + k
        vk = beta_k * vk + ik
        sk = (vk >= 1.0).astype(f32)
        vk = vk * (1.0 - sk)

        iv = alpha_v * iv + v
        vv = beta_v * vv + iv
        sv = (vv >= 1.0).astype(f32)
        vv = vv * (1.0 - sv)

        sq_b = (sq * mask).astype(jnp.bfloat16)
        sk_b = (sk * mask).astype(jnp.bfloat16)
        vsum_ref[...] += sv * mask

        for h in range(H):
            hs = slice(h * HEAD_DIM, (h + 1) * HEAD_DIM)
            scores_ref[h] += jax.lax.dot_general(
                sq_b[:, hs], sk_b[:, hs],
                (((1,), (1,)), ((), ())),
                preferred_element_type=f32)
        return (vq, iq, vk, ik, vv, iv)

    z = jnp.zeros((S, D), f32)
    jax.lax.fori_loop(0, T_MAX, step, (z, z, z, z, z, z), unroll=False)

    v_mean = vsum_ref[...] * (1.0 / T_MAX)
    for h in range(H):
        hs = slice(h * HEAD_DIM, (h + 1) * HEAD_DIM)
        s = scores_ref[h] * SCALE
        m = jnp.max(s, axis=1, keepdims=True)
        e = jnp.exp(s - m)
        attn = e / jnp.sum(e, axis=1, keepdims=True)
        ctx_ref[:, hs] = jnp.dot(attn, v_mean[:, hs],
                                 preferred_element_type=f32)

    out_ref[0] = _nt(ctx_ref[...], ow_ref[...]) + ob_ref[...]


@jax.jit
def kernel(x, q_w, k_w, v_w, out_w, out_b,
           g_w1, g_b1, g_w2, g_b2, g_w3, g_b3,
           c_w1, c_b1, c_w2, c_b2,
           alpha_q, beta_q, alpha_k, beta_k, alpha_v, beta_v):
    f32 = jnp.float32
    ab = jnp.stack([alpha_q, beta_q, alpha_k, beta_k,
                    alpha_v, beta_v]).reshape(1, 6).astype(f32)
    gw3p = jnp.zeros((128, 32), f32).at[0].set(g_w3[0])
    cw2p = jnp.zeros((128, 32), f32).at[0].set(c_w2[0])
    full = lambda shape: pl.BlockSpec(shape, lambda b: (0,) * len(shape))
    out = pl.pallas_call(
        _fused_kernel,
        grid=(B,),
        in_specs=[
            pl.BlockSpec((1, S, D), lambda b: (b, 0, 0)),
            full((D, D)), full((D, D)), full((D, D)),
            full((D, D)), full((1, D)),
            full((64, D)), full((1, 64)),
            full((32, 64)), full((1, 32)),
            full((128, 32)), full((1, 1)),
            full((32, D)), full((1, 32)),
            full((128, 32)), full((1, 1)),
            pl.BlockSpec(memory_space=pltpu.SMEM),
        ],
        out_specs=pl.BlockSpec((1, S, D), lambda b: (b, 0, 0)),
        scratch_shapes=[
            pltpu.VMEM((H, S, S), f32),
            pltpu.VMEM((S, D), f32),
            pltpu.VMEM((S, D), f32),
        ],
        out_shape=jax.ShapeDtypeStruct((B, S, D), f32),
    )(x,
      q_w, k_w, v_w, out_w, out_b.reshape(1, D),
      g_w1, g_b1.reshape(1, 64), g_w2, g_b2.reshape(1, 32),
      gw3p, g_b3.reshape(1, 1),
      c_w1, c_b1.reshape(1, 32), cw2p, c_b2.reshape(1, 1),
      ab)
    return out


# speedup vs baseline: 8.8680x; 1.7422x over previous
import jax
import jax.numpy as jnp
from jax.experimental import pallas as pl
from jax.experimental.pallas import tpu as pltpu

B, S, D = 4, 512, 256
H, T_MAX = 8, 20
HEAD_DIM = D // H
KDIM = T_MAX * HEAD_DIM
SCALE = HEAD_DIM ** -0.5


def _nt(a, b):
    return jax.lax.dot_general(a, b, (((1,), (1,)), ((), ())),
                               preferred_element_type=jnp.float32)


def _fused_kernel(x_ref, qw_ref, kw_ref, vw_ref, ow_ref, ob_ref,
                  gw1_ref, gb1_ref, gw2_ref, gb2_ref, gw3_ref, gb3_ref,
                  cw1_ref, cb1_ref, cw2_ref, cb2_ref,
                  ab_ref,
                  out_ref,
                  qbuf_ref, kbuf_ref, vsum_ref, ctx_ref):
    f32 = jnp.float32
    bf16 = jnp.bfloat16
    xb = x_ref[0]

    q = _nt(xb, qw_ref[...])
    k = _nt(xb, kw_ref[...])
    v = _nt(xb, vw_ref[...])

    h1 = jnp.maximum(_nt(xb, gw1_ref[...]) + gb1_ref[...], 0.0)
    h2 = jnp.maximum(_nt(h1, gw2_ref[...]) + gb2_ref[...], 0.0)
    g3 = _nt(h2, gw3_ref[...])[:, 0:1] + gb3_ref[0, 0]
    gate = jax.nn.sigmoid(g3)
    hc = jnp.maximum(_nt(xb, cw1_ref[...]) + cb1_ref[...], 0.0)
    c2 = _nt(hc, cw2_ref[...])[:, 0:1] + cb2_ref[0, 0]
    comp = jax.nn.sigmoid(c2)
    combined = 0.7 * gate + 0.3 * comp
    t_lim = jnp.clip(jnp.ceil(combined * T_MAX), 1.0, float(T_MAX))

    alpha_q = ab_ref[0, 0]
    beta_q = ab_ref[0, 1]
    alpha_k = ab_ref[0, 2]
    beta_k = ab_ref[0, 3]
    alpha_v = ab_ref[0, 4]
    beta_v = ab_ref[0, 5]

    vq = iq = vk = ik = vv = iv = jnp.zeros((S, D), f32)
    vsum = jnp.zeros((S, D), f32)
    for t in range(T_MAX):
        mask = (float(t) < t_lim).astype(f32)

        iq = alpha_q * iq + q
        vq = beta_q * vq + iq
        sq = (vq >= 1.0).astype(f32)
        vq = vq * (1.0 - sq)

        ik = alpha_k * ik + k
        vk = beta_k * vk + ik
        sk = (vk >= 1.0).astype(f32)
        vk = vk * (1.0 - sk)

        iv = alpha_v * iv + v
        vv = beta_v * vv + iv
        sv = (vv >= 1.0).astype(f32)
        vv = vv * (1.0 - sv)

        sq_b = (sq * mask).astype(bf16)
        sk_b = (sk * mask).astype(bf16)
        vsum = vsum + sv * mask

        ts = slice(t * HEAD_DIM, (t + 1) * HEAD_DIM)
        for h in range(H):
            hs = slice(h * HEAD_DIM, (h + 1) * HEAD_DIM)
            qbuf_ref[h, :, ts] = sq_b[:, hs]
            kbuf_ref[h, :, ts] = sk_b[:, hs]

    vsum_ref[...] = vsum

    v_mean = vsum_ref[...] * (1.0 / T_MAX)
    for h in range(H):
        hs = slice(h * HEAD_DIM, (h + 1) * HEAD_DIM)
        s = jax.lax.dot_general(
            qbuf_ref[h], kbuf_ref[h],
            (((1,), (1,)), ((), ())),
            preferred_element_type=f32) * SCALE
        m = jnp.max(s, axis=1, keepdims=True)
        e = jnp.exp(s - m)
        attn = e / jnp.sum(e, axis=1, keepdims=True)
        ctx_ref[:, hs] = jnp.dot(attn, v_mean[:, hs],
                                 preferred_element_type=f32)

    out_ref[0] = _nt(ctx_ref[...], ow_ref[...]) + ob_ref[...]


@jax.jit
def kernel(x, q_w, k_w, v_w, out_w, out_b,
           g_w1, g_b1, g_w2, g_b2, g_w3, g_b3,
           c_w1, c_b1, c_w2, c_b2,
           alpha_q, beta_q, alpha_k, beta_k, alpha_v, beta_v):
    f32 = jnp.float32
    ab = jnp.stack([alpha_q, beta_q, alpha_k, beta_k,
                    alpha_v, beta_v]).reshape(1, 6).astype(f32)
    gw3p = jnp.zeros((128, 32), f32).at[0].set(g_w3[0])
    cw2p = jnp.zeros((128, 32), f32).at[0].set(c_w2[0])
    full = lambda shape: pl.BlockSpec(shape, lambda b: (0,) * len(shape))
    out = pl.pallas_call(
        _fused_kernel,
        grid=(B,),
        in_specs=[
            pl.BlockSpec((1, S, D), lambda b: (b, 0, 0)),
            full((D, D)), full((D, D)), full((D, D)),
            full((D, D)), full((1, D)),
            full((64, D)), full((1, 64)),
            full((32, 64)), full((1, 32)),
            full((128, 32)), full((1, 1)),
            full((32, D)), full((1, 32)),
            full((128, 32)), full((1, 1)),
            pl.BlockSpec(memory_space=pltpu.SMEM),
        ],
        out_specs=pl.BlockSpec((1, S, D), lambda b: (b, 0, 0)),
        scratch_shapes=[
            pltpu.VMEM((H, S, KDIM), jnp.bfloat16),
            pltpu.VMEM((H, S, KDIM), jnp.bfloat16),
            pltpu.VMEM((S, D), f32),
            pltpu.VMEM((S, D), f32),
        ],
        out_shape=jax.ShapeDtypeStruct((B, S, D), f32),
    )(x,
      q_w, k_w, v_w, out_w, out_b.reshape(1, D),
      g_w1, g_b1.reshape(1, 64), g_w2, g_b2.reshape(1, 32),
      gw3p, g_b3.reshape(1, 1),
      c_w1, c_b1.reshape(1, 32), cw2p, c_b2.reshape(1, 1),
      ab)
    return out


# speedup vs baseline: 9.4727x; 1.0682x over previous
import jax
import jax.numpy as jnp
from jax.experimental import pallas as pl
from jax.experimental.pallas import tpu as pltpu

B, S, D = 4, 512, 256
H, T_MAX = 8, 20
HEAD_DIM = D // H
KDIM = T_MAX * HEAD_DIM
SCALE = HEAD_DIM ** -0.5
TR = 64


def _nt(a, b):
    return jax.lax.dot_general(a, b, (((1,), (1,)), ((), ())),
                               preferred_element_type=jnp.float32)


def _fused_kernel(x_ref, qw_ref, kw_ref, vw_ref, ow_ref, ob_ref,
                  gw1_ref, gb1_ref, gw2_ref, gb2_ref, gw3_ref, gb3_ref,
                  cw1_ref, cb1_ref, cw2_ref, cb2_ref,
                  ab_ref,
                  out_ref,
                  qbuf_ref, kbuf_ref, vsum_ref, ctx_ref):
    f32 = jnp.float32
    bf16 = jnp.bfloat16
    xb = x_ref[0]

    q = _nt(xb, qw_ref[...])
    k = _nt(xb, kw_ref[...])
    v = _nt(xb, vw_ref[...])

    h1 = jnp.maximum(_nt(xb, gw1_ref[...]) + gb1_ref[...], 0.0)
    h2 = jnp.maximum(_nt(h1, gw2_ref[...]) + gb2_ref[...], 0.0)
    g3 = _nt(h2, gw3_ref[...])[:, 0:1] + gb3_ref[0, 0]
    gate = jax.nn.sigmoid(g3)
    hc = jnp.maximum(_nt(xb, cw1_ref[...]) + cb1_ref[...], 0.0)
    c2 = _nt(hc, cw2_ref[...])[:, 0:1] + cb2_ref[0, 0]
    comp = jax.nn.sigmoid(c2)
    combined = 0.7 * gate + 0.3 * comp
    t_lim = jnp.clip(jnp.ceil(combined * T_MAX), 1.0, float(T_MAX))

    alpha_q = ab_ref[0, 0]
    beta_q = ab_ref[0, 1]
    alpha_k = ab_ref[0, 2]
    beta_k = ab_ref[0, 3]
    alpha_v = ab_ref[0, 4]
    beta_v = ab_ref[0, 5]

    for r in range(S // TR):
        rs = slice(r * TR, (r + 1) * TR)
        qt, kt, vt = q[rs], k[rs], v[rs]
        z = jnp.zeros((TR, D), f32)
        tlb = t_lim[rs] + z
        vq = iq = vk = ik = vv = iv = z
        vsum = z
        for t in range(T_MAX):
            mf = tlb > float(t)

            iq = alpha_q * iq + qt
            vq = beta_q * vq + iq
            cq = vq >= 1.0
            vq = jnp.where(cq, 0.0, vq)

            ik = alpha_k * ik + kt
            vk = beta_k * vk + ik
            ck = vk >= 1.0
            vk = jnp.where(ck, 0.0, vk)

            iv = alpha_v * iv + vt
            vv = beta_v * vv + iv
            cv = vv >= 1.0
            vv = jnp.where(cv, 0.0, vv)

            sq_b = jnp.where(cq & mf, 1.0, 0.0).astype(bf16)
            sk_b = jnp.where(ck & mf, 1.0, 0.0).astype(bf16)
            vsum = vsum + jnp.where(cv & mf, 1.0, 0.0)

            ts = slice(t * HEAD_DIM, (t + 1) * HEAD_DIM)
            for h in range(H):
                hs = slice(h * HEAD_DIM, (h + 1) * HEAD_DIM)
                qbuf_ref[h, rs, ts] = sq_b[:, hs]
                kbuf_ref[h, rs, ts] = sk_b[:, hs]
        vsum_ref[rs, :] = vsum

    v_mean = vsum_ref[...] * (1.0 / T_MAX)
    ones_nt = jnp.ones((8, S), f32)
    for h in range(H):
        hs = slice(h * HEAD_DIM, (h + 1) * HEAD_DIM)
        s = jax.lax.dot_general(
            qbuf_ref[h], kbuf_ref[h],
            (((1,), (1,)), ((), ())),
            preferred_element_type=f32)
        m = jnp.max(s, axis=1, keepdims=True)
        e = jnp.exp((s - m) * SCALE)
        denom = _nt(e, ones_nt)[:, 0:1]
        ctx = jnp.dot(e, v_mean[:, hs], preferred_element_type=f32)
        ctx_ref[:, hs] = ctx * (1.0 / denom)

    out_ref[0] = _nt(ctx_ref[...], ow_ref[...]) + ob_ref[...]


@jax.jit
def kernel(x, q_w, k_w, v_w, out_w, out_b,
           g_w1, g_b1, g_w2, g_b2, g_w3, g_b3,
           c_w1, c_b1, c_w2, c_b2,
           alpha_q, beta_q, alpha_k, beta_k, alpha_v, beta_v):
    f32 = jnp.float32
    ab = jnp.stack([alpha_q, beta_q, alpha_k, beta_k,
                    alpha_v, beta_v]).reshape(1, 6).astype(f32)
    gw3p = jnp.zeros((128, 32), f32).at[0].set(g_w3[0])
    cw2p = jnp.zeros((128, 32), f32).at[0].set(c_w2[0])
    full = lambda shape: pl.BlockSpec(shape, lambda b: (0,) * len(shape))
    out = pl.pallas_call(
        _fused_kernel,
        grid=(B,),
        in_specs=[
            pl.BlockSpec((1, S, D), lambda b: (b, 0, 0)),
            full((D, D)), full((D, D)), full((D, D)),
            full((D, D)), full((1, D)),
            full((64, D)), full((1, 64)),
            full((32, 64)), full((1, 32)),
            full((128, 32)), full((1, 1)),
            full((32, D)), full((1, 32)),
            full((128, 32)), full((1, 1)),
            pl.BlockSpec(memory_space=pltpu.SMEM),
        ],
        out_specs=pl.BlockSpec((1, S, D), lambda b: (b, 0, 0)),
        scratch_shapes=[
            pltpu.VMEM((H, S, KDIM), jnp.bfloat16),
            pltpu.VMEM((H, S, KDIM), jnp.bfloat16),
            pltpu.VMEM((S, D), f32),
            pltpu.VMEM((S, D), f32),
        ],
        out_shape=jax.ShapeDtypeStruct((B, S, D), f32),
    )(x,
      q_w, k_w, v_w, out_w, out_b.reshape(1, D),
      g_w1, g_b1.reshape(1, 64), g_w2, g_b2.reshape(1, 32),
      gw3p, g_b3.reshape(1, 1),
      c_w1, c_b1.reshape(1, 32), cw2p, c_b2.reshape(1, 1),
      ab)
    return out


# speedup vs baseline: 9.5299x; 1.0060x over previous
import jax
import jax.numpy as jnp
from jax.experimental import pallas as pl
from jax.experimental.pallas import tpu as pltpu

B, S, D = 4, 512, 256
H, T_MAX = 8, 20
HEAD_DIM = D // H
KDIM = T_MAX * HEAD_DIM
SCALE = HEAD_DIM ** -0.5
TR = 64


def _nt(a, b):
    return jax.lax.dot_general(a, b, (((1,), (1,)), ((), ())),
                               preferred_element_type=jnp.float32)


def _fused_kernel(x_ref, qw_ref, kw_ref, vw_ref, ow_ref, ob_ref,
                  gw1_ref, gb1_ref, gw2_ref, gb2_ref, gw3_ref, gb3_ref,
                  cw1_ref, cb1_ref, cw2_ref, cb2_ref,
                  ab_ref,
                  out_ref,
                  qbuf_ref, kbuf_ref, vsum_ref, ctx_ref):
    f32 = jnp.float32
    bf16 = jnp.bfloat16
    xb = x_ref[0]

    q = _nt(xb, qw_ref[...])
    k = _nt(xb, kw_ref[...])
    v = _nt(xb, vw_ref[...])

    h1 = jnp.maximum(_nt(xb, gw1_ref[...]) + gb1_ref[...], 0.0)
    h2 = jnp.maximum(_nt(h1, gw2_ref[...]) + gb2_ref[...], 0.0)
    g3 = _nt(h2, gw3_ref[...])[:, 0:1] + gb3_ref[0, 0]
    gate = jax.nn.sigmoid(g3)
    hc = jnp.maximum(_nt(xb, cw1_ref[...]) + cb1_ref[...], 0.0)
    c2 = _nt(hc, cw2_ref[...])[:, 0:1] + cb2_ref[0, 0]
    comp = jax.nn.sigmoid(c2)
    combined = 0.7 * gate + 0.3 * comp
    t_lim = jnp.clip(jnp.ceil(combined * T_MAX), 1.0, float(T_MAX))

    alpha_q = ab_ref[0, 0]
    beta_q = ab_ref[0, 1]
    alpha_k = ab_ref[0, 2]
    beta_k = ab_ref[0, 3]
    alpha_v = ab_ref[0, 4]
    beta_v = ab_ref[0, 5]

    for r in range(S // TR):
        rs = slice(r * TR, (r + 1) * TR)
        qt, kt, vt = q[rs], k[rs], v[rs]
        z = jnp.zeros((TR, D), f32)
        tlb = t_lim[rs] + z
        vq = iq = vk = ik = vv = iv = z
        vsum = z
        for t in range(T_MAX):
            mf = tlb > float(t)

            iq = alpha_q * iq + qt
            vq = beta_q * vq + iq
            cq = vq >= 1.0
            vq = jnp.where(cq, 0.0, vq)

            ik = alpha_k * ik + kt
            vk = beta_k * vk + ik
            ck = vk >= 1.0
            vk = jnp.where(ck, 0.0, vk)

            iv = alpha_v * iv + vt
            vv = beta_v * vv + iv
            cv = vv >= 1.0
            vv = jnp.where(cv, 0.0, vv)

            sq_b = jnp.where(cq & mf, 1.0, 0.0).astype(bf16)
            sk_b = jnp.where(ck & mf, 1.0, 0.0).astype(bf16)
            vsum = vsum + jnp.where(cv & mf, 1.0, 0.0)

            ts = slice(t * HEAD_DIM, (t + 1) * HEAD_DIM)
            for h in range(H):
                hs = slice(h * HEAD_DIM, (h + 1) * HEAD_DIM)
                qbuf_ref[h, rs, ts] = sq_b[:, hs]
                kbuf_ref[h, rs, ts] = sk_b[:, hs]
        vsum_ref[rs, :] = vsum

    v_mean = vsum_ref[...] * (1.0 / T_MAX)
    ones_nt = jnp.ones((8, S), f32)
    for h in range(H):
        hs = slice(h * HEAD_DIM, (h + 1) * HEAD_DIM)
        s = jax.lax.dot_general(
            qbuf_ref[h], kbuf_ref[h],
            (((1,), (1,)), ((), ())),
            preferred_element_type=f32)
        m = jnp.max(s, axis=1, keepdims=True)
        e = jnp.exp((s - m) * SCALE)
        denom = _nt(e, ones_nt)[:, 0:1]
        ctx = jnp.dot(e, v_mean[:, hs], preferred_element_type=f32)
        ctx_ref[:, hs] = ctx * (1.0 / denom)

    out_ref[0] = _nt(ctx_ref[...], ow_ref[...]) + ob_ref[...]


@jax.jit
def kernel(x, q_w, k_w, v_w, out_w, out_b,
           g_w1, g_b1, g_w2, g_b2, g_w3, g_b3,
           c_w1, c_b1, c_w2, c_b2,
           alpha_q, beta_q, alpha_k, beta_k, alpha_v, beta_v):
    f32 = jnp.float32
    ab = jnp.stack([alpha_q, beta_q, alpha_k, beta_k,
                    alpha_v, beta_v]).reshape(1, 6).astype(f32)
    gw3p = jnp.zeros((128, 32), f32).at[0].set(g_w3[0])
    cw2p = jnp.zeros((128, 32), f32).at[0].set(c_w2[0])
    full = lambda shape: pl.BlockSpec(shape, lambda b: (0,) * len(shape))
    out = pl.pallas_call(
        _fused_kernel,
        grid=(B,),
        in_specs=[
            pl.BlockSpec((1, S, D), lambda b: (b, 0, 0)),
            full((D, D)), full((D, D)), full((D, D)),
            full((D, D)), full((1, D)),
            full((64, D)), full((1, 64)),
            full((32, 64)), full((1, 32)),
            full((128, 32)), full((1, 1)),
            full((32, D)), full((1, 32)),
            full((128, 32)), full((1, 1)),
            pl.BlockSpec(memory_space=pltpu.SMEM),
        ],
        out_specs=pl.BlockSpec((1, S, D), lambda b: (b, 0, 0)),
        scratch_shapes=[
            pltpu.VMEM((H, S, KDIM), jnp.bfloat16),
            pltpu.VMEM((H, S, KDIM), jnp.bfloat16),
            pltpu.VMEM((S, D), f32),
            pltpu.VMEM((S, D), f32),
        ],
        out_shape=jax.ShapeDtypeStruct((B, S, D), f32),
        compiler_params=pltpu.CompilerParams(
            dimension_semantics=("parallel",)),
    )(x,
      q_w, k_w, v_w, out_w, out_b.reshape(1, D),
      g_w1, g_b1.reshape(1, 64), g_w2, g_b2.reshape(1, 32),
      gw3p, g_b3.reshape(1, 1),
      c_w1, c_b1.reshape(1, 32), cw2p, c_b2.reshape(1, 1),
      ab)
    return out


# speedup vs baseline: 14.0716x; 1.4766x over previous
import jax
import jax.numpy as jnp
from jax.experimental import pallas as pl
from jax.experimental.pallas import tpu as pltpu

B, S, D = 4, 512, 256
H, T_MAX = 8, 20
HEAD_DIM = D // H
KDIM = T_MAX * HEAD_DIM
SCALE = HEAD_DIM ** -0.5
TR = 64


def _nt(a, b):
    return jax.lax.dot_general(a, b, (((1,), (1,)), ((), ())),
                               preferred_element_type=jnp.float32)


def _fused_kernel(x_ref, qw_ref, kw_ref, vw_ref, ow_ref, ob_ref,
                  gw1_ref, gb1_ref, gw2_ref, gb2_ref, gw3_ref, gb3_ref,
                  cw1_ref, cb1_ref, cw2_ref, cb2_ref,
                  ab_ref,
                  out_ref,
                  qbuf_ref, kbuf_ref, vsum_ref, ctx_ref):
    f32 = jnp.float32
    bf16 = jnp.bfloat16
    xb = x_ref[0]

    q = _nt(xb, qw_ref[...])
    k = _nt(xb, kw_ref[...])
    v = _nt(xb, vw_ref[...])

    h1 = jnp.maximum(_nt(xb, gw1_ref[...]) + gb1_ref[...], 0.0)
    h2 = jnp.maximum(_nt(h1, gw2_ref[...]) + gb2_ref[...], 0.0)
    g3 = _nt(h2, gw3_ref[...])[:, 0:1] + gb3_ref[0, 0]
    gate = jax.nn.sigmoid(g3)
    hc = jnp.maximum(_nt(xb, cw1_ref[...]) + cb1_ref[...], 0.0)
    c2 = _nt(hc, cw2_ref[...])[:, 0:1] + cb2_ref[0, 0]
    comp = jax.nn.sigmoid(c2)
    combined = 0.7 * gate + 0.3 * comp
    t_lim = jnp.clip(jnp.ceil(combined * T_MAX), 1.0, float(T_MAX))

    alpha_q = ab_ref[0, 0]
    beta_q = ab_ref[0, 1]
    alpha_k = ab_ref[0, 2]
    beta_k = ab_ref[0, 3]
    alpha_v = ab_ref[0, 4]
    beta_v = ab_ref[0, 5]

    for r in range(S // TR):
        rs = slice(r * TR, (r + 1) * TR)
        qt, kt, vt = q[rs], k[rs], v[rs]
        z = jnp.zeros((TR, D), f32)
        tlb = t_lim[rs] + z
        vq = iq = vk = ik = vv = iv = z
        vsum = z
        for t in range(T_MAX):
            mf = tlb > float(t)

            iq = alpha_q * iq + qt
            vq = beta_q * vq + iq
            cq = vq >= 1.0
            vq = jnp.where(cq, 0.0, vq)

            ik = alpha_k * ik + kt
            vk = beta_k * vk + ik
            ck = vk >= 1.0
            vk = jnp.where(ck, 0.0, vk)

            iv = alpha_v * iv + vt
            vv = beta_v * vv + iv
            cv = vv >= 1.0
            vv = jnp.where(cv, 0.0, vv)

            sq_b = jnp.where(cq & mf, 1, 0).astype(jnp.int8)
            sk_b = jnp.where(ck & mf, 1, 0).astype(jnp.int8)
            vsum = vsum + jnp.where(cv & mf, 1.0, 0.0)

            ts = slice(t * HEAD_DIM, (t + 1) * HEAD_DIM)
            for h in range(H):
                hs = slice(h * HEAD_DIM, (h + 1) * HEAD_DIM)
                qbuf_ref[h, rs, ts] = sq_b[:, hs]
                kbuf_ref[h, rs, ts] = sk_b[:, hs]
        vsum_ref[rs, :] = vsum

    v_mean = vsum_ref[...] * (1.0 / T_MAX)
    ones_nt = jnp.ones((8, S), f32)
    for h in range(H):
        hs = slice(h * HEAD_DIM, (h + 1) * HEAD_DIM)
        s = jax.lax.dot_general(
            qbuf_ref[h], kbuf_ref[h],
            (((1,), (1,)), ((), ())),
            preferred_element_type=jnp.int32).astype(f32)
        m = jnp.max(s, axis=1, keepdims=True)
        e = jnp.exp((s - m) * SCALE)
        denom = _nt(e, ones_nt)[:, 0:1]
        ctx = jnp.dot(e, v_mean[:, hs], preferred_element_type=f32)
        ctx_ref[:, hs] = ctx * (1.0 / denom)

    out_ref[0] = _nt(ctx_ref[...], ow_ref[...]) + ob_ref[...]


@jax.jit
def kernel(x, q_w, k_w, v_w, out_w, out_b,
           g_w1, g_b1, g_w2, g_b2, g_w3, g_b3,
           c_w1, c_b1, c_w2, c_b2,
           alpha_q, beta_q, alpha_k, beta_k, alpha_v, beta_v):
    f32 = jnp.float32
    ab = jnp.stack([alpha_q, beta_q, alpha_k, beta_k,
                    alpha_v, beta_v]).reshape(1, 6).astype(f32)
    gw3p = jnp.zeros((128, 32), f32).at[0].set(g_w3[0])
    cw2p = jnp.zeros((128, 32), f32).at[0].set(c_w2[0])
    full = lambda shape: pl.BlockSpec(shape, lambda b: (0,) * len(shape))
    out = pl.pallas_call(
        _fused_kernel,
        grid=(B,),
        in_specs=[
            pl.BlockSpec((1, S, D), lambda b: (b, 0, 0)),
            full((D, D)), full((D, D)), full((D, D)),
            full((D, D)), full((1, D)),
            full((64, D)), full((1, 64)),
            full((32, 64)), full((1, 32)),
            full((128, 32)), full((1, 1)),
            full((32, D)), full((1, 32)),
            full((128, 32)), full((1, 1)),
            pl.BlockSpec(memory_space=pltpu.SMEM),
        ],
        out_specs=pl.BlockSpec((1, S, D), lambda b: (b, 0, 0)),
        scratch_shapes=[
            pltpu.VMEM((H, S, KDIM), jnp.int8),
            pltpu.VMEM((H, S, KDIM), jnp.int8),
            pltpu.VMEM((S, D), f32),
            pltpu.VMEM((S, D), f32),
        ],
        out_shape=jax.ShapeDtypeStruct((B, S, D), f32),
        compiler_params=pltpu.CompilerParams(
            dimension_semantics=("parallel",)),
    )(x,
      q_w, k_w, v_w, out_w, out_b.reshape(1, D),
      g_w1, g_b1.reshape(1, 64), g_w2, g_b2.reshape(1, 32),
      gw3p, g_b3.reshape(1, 1),
      c_w1, c_b1.reshape(1, 32), cw2p, c_b2.reshape(1, 1),
      ab)
    return out
